# Initial kernel scaffold; baseline (speedup 1.0000x reference)
import jax
import jax.numpy as jnp
from jax.experimental import pallas as pl


def kernel(x, edge_index, W1, b1, W2, b2):
    raise NotImplementedError("write your pallas kernel here")


# speedup vs baseline: 27.6850x; 27.6850x over previous
import functools

import jax
import jax.numpy as jnp
from jax import lax
from jax.experimental import pallas as pl
from jax.experimental.pallas import tpu as pltpu
from jax.experimental.pallas import tpu_sc as plsc

N_NODES = 10000
N_EDGES = 320000
F_IN = 128
F_HID = 16

NC = 2
NS = 16
NW = NC * NS
CHUNK = 80
ROWS_PER_TILE = N_EDGES // NW // CHUNK
NPAD = 10240
ZSLICE = NPAD // NS

_MESH = plsc.VectorSubcoreMesh(core_axis_name="c", subcore_axis_name="s")


def _zero_f32(ref, n):
  for i in range(n // 16):
    ref[pl.ds(i * 16, 16)] = jnp.zeros((16,), jnp.float32)


@functools.partial(
    pl.kernel,
    out_type=jax.ShapeDtypeStruct((NC, NS, ZSLICE), jnp.float32),
    mesh=_MESH,
    scratch_types=[
        pltpu.VMEM((ROWS_PER_TILE, CHUNK), jnp.int32),
        pltpu.VMEM((CHUNK,), jnp.float32),
        pltpu.VMEM((ZSLICE,), jnp.float32),
        pltpu.VMEM_SHARED((NPAD,), jnp.float32),
    ],
)
def _sc_degree(dst_hbm, out_hbm, idx_v, ones_v, zero_v, shared):
  cid = lax.axis_index("c")
  sid = lax.axis_index("s")
  wid = sid * NC + cid

  _zero_f32(zero_v, ZSLICE)
  pltpu.sync_copy(zero_v, shared.at[pl.ds(sid * ZSLICE, ZSLICE)])
  for i in range(CHUNK // 16):
    ones_v[pl.ds(i * 16, 16)] = jnp.ones((16,), jnp.float32)
  pltpu.sync_copy(dst_hbm.at[wid], idx_v)
  plsc.subcore_barrier()

  def body(j, carry):
    pltpu.sync_copy(ones_v, shared.at[idx_v.at[j]], add=True)
    return carry

  lax.fori_loop(0, ROWS_PER_TILE, body, 0)
  plsc.subcore_barrier()
  pltpu.sync_copy(shared.at[pl.ds(sid * ZSLICE, ZSLICE)], out_hbm.at[cid, sid])


@functools.partial(
    pl.kernel,
    out_type=jax.ShapeDtypeStruct((NC, NPAD, F_HID), jnp.float32),
    mesh=_MESH,
    scratch_types=[
        pltpu.VMEM((ROWS_PER_TILE, CHUNK), jnp.int32),
        pltpu.VMEM((ROWS_PER_TILE, CHUNK), jnp.int32),
        pltpu.VMEM((CHUNK, F_HID), jnp.float32),
        pltpu.VMEM((160, F_HID), jnp.float32),
        pltpu.VMEM_SHARED((NPAD, F_HID), jnp.float32),
        pltpu.SemaphoreType.DMA,
    ],
    compiler_params=pltpu.CompilerParams(use_tc_tiling_on_sc=False),
)
def _sc_aggregate(table_hbm, src_hbm, dst_hbm, out_hbm,
                  sidx_v, didx_v, rows_v, zbuf_v, shared, gsem):
  cid = lax.axis_index("c")
  sid = lax.axis_index("s")
  wid = sid * NC + cid

  for i in range(160):
    zbuf_v[i] = jnp.zeros((F_HID,), jnp.float32)
  for i in range(ZSLICE // 160):
    pltpu.sync_copy(zbuf_v, shared.at[pl.ds(sid * ZSLICE + i * 160, 160)])
  pltpu.sync_copy(src_hbm.at[wid], sidx_v)
  pltpu.sync_copy(dst_hbm.at[wid], didx_v)
  plsc.subcore_barrier()

  def body(j, carry):
    pltpu.async_copy(table_hbm.at[sidx_v.at[j]], rows_v, gsem).wait()
    pltpu.sync_copy(rows_v, shared.at[didx_v.at[j]], add=True)
    return carry

  lax.fori_loop(0, ROWS_PER_TILE, body, 0)
  plsc.subcore_barrier()
  pltpu.sync_copy(shared.at[pl.ds(sid * ZSLICE, ZSLICE)],
                  out_hbm.at[cid].at[pl.ds(sid * ZSLICE, ZSLICE)])


def _dinv_of(degp_ref):
  deg = degp_ref[:, 0:1] + degp_ref[:, 1:2] + 1.0
  return lax.rsqrt(deg)


def _tc1_body(x_ref, w1_ref, degp_ref, xw_ref, t1_ref):
  xw = jnp.dot(x_ref[...], w1_ref[...], preferred_element_type=jnp.float32)
  xw_ref[...] = xw
  t1_ref[...] = xw * _dinv_of(degp_ref)


def _tc2_body(degp_ref, xw_ref, a0_ref, a1_ref, b1_ref, w2_ref,
              hw_ref, t2_ref):
  dinv = _dinv_of(degp_ref)
  h = dinv * (a0_ref[...] + a1_ref[...]) + dinv * dinv * xw_ref[...]
  h = jnp.maximum(h + b1_ref[...], 0.0)
  hw = jnp.dot(h, w2_ref[...], preferred_element_type=jnp.float32)
  hw_ref[...] = hw
  t2_ref[...] = hw * dinv


def _tc3_body(degp_ref, hw_ref, a0_ref, a1_ref, b2_ref, out_ref):
  dinv = _dinv_of(degp_ref)
  out = dinv * (a0_ref[...] + a1_ref[...]) + dinv * dinv * hw_ref[...]
  out_ref[...] = jnp.maximum(out + b2_ref[...], 0.0)


_GRID = 10
_RB = N_NODES // _GRID


def _row_spec(width):
  return pl.BlockSpec((_RB, width), lambda i: (i, 0))


def _full_spec(shape):
  return pl.BlockSpec(shape, lambda i: tuple(0 for _ in shape))


def kernel(x, edge_index, W1, b1, W2, b2):
  src3d = edge_index[0].reshape(NW, ROWS_PER_TILE, CHUNK)
  dst3d = edge_index[1].reshape(NW, ROWS_PER_TILE, CHUNK)

  degp = _sc_degree(dst3d).reshape(NC, NPAD)
  degp_t = degp.T[:N_NODES]

  xw1, table1 = pl.pallas_call(
      _tc1_body,
      grid=(_GRID,),
      in_specs=[_row_spec(F_IN), _full_spec((F_IN, F_HID)), _row_spec(2)],
      out_specs=[_row_spec(F_HID), _row_spec(F_HID)],
      out_shape=[jax.ShapeDtypeStruct((N_NODES, F_HID), jnp.float32)] * 2,
  )(x, W1, degp_t)

  agg1 = _sc_aggregate(table1, src3d, dst3d)

  hw2, table2 = pl.pallas_call(
      _tc2_body,
      grid=(_GRID,),
      in_specs=[_row_spec(2), _row_spec(F_HID), _row_spec(F_HID),
                _row_spec(F_HID), _full_spec((1, F_HID)),
                _full_spec((F_HID, F_HID))],
      out_specs=[_row_spec(F_HID), _row_spec(F_HID)],
      out_shape=[jax.ShapeDtypeStruct((N_NODES, F_HID), jnp.float32)] * 2,
  )(degp_t, xw1, agg1[0, :N_NODES], agg1[1, :N_NODES],
    b1.reshape(1, F_HID), W2)

  agg2 = _sc_aggregate(table2, src3d, dst3d)

  out = pl.pallas_call(
      _tc3_body,
      grid=(_GRID,),
      in_specs=[_row_spec(2), _row_spec(F_HID), _row_spec(F_HID),
                _row_spec(F_HID), _full_spec((1, F_HID))],
      out_specs=_row_spec(F_HID),
      out_shape=jax.ShapeDtypeStruct((N_NODES, F_HID), jnp.float32),
  )(degp_t, hw2, agg2[0, :N_NODES], agg2[1, :N_NODES], b2.reshape(1, F_HID))

  return out
